# Initial kernel scaffold; baseline (speedup 1.0000x reference)
import jax
import jax.numpy as jnp
from jax.experimental import pallas as pl


def kernel(x, edge_index, edge_attr, W1, b1, W2, b2):
    raise NotImplementedError("write your pallas kernel here")


# speedup vs baseline: 5.0208x; 5.0208x over previous
import functools

import jax
import jax.numpy as jnp
from jax import lax
from jax.experimental import pallas as pl
from jax.experimental.pallas import tpu as pltpu
from jax.experimental.pallas import tpu_sc as plsc

N = 10000
NP = 10240
DF = 256
HALF = 128
E = 160000
NS = 16
L = 16
NPT = NP // NS
EPT = E // NS
KE = 80
CH2 = 64

_i32 = jnp.int32
_f32 = jnp.float32


def _sigmoid16(v):
    return 1.0 / (1.0 + jnp.exp(-v))


def _zero_vec(ref, n):
    z = jnp.zeros((L,), _f32)

    def body(i, _):
        ref[pl.ds(i * L, L)] = z
        return 0

    lax.fori_loop(0, n // L, body, 0)


@functools.partial(
    pl.kernel,
    out_type=jax.ShapeDtypeStruct((NP,), _f32),
    mesh=plsc.VectorSubcoreMesh(
        core_axis_name="c", subcore_axis_name="s", num_cores=1),
    compiler_params=pltpu.CompilerParams(needs_layout_passes=False),
    scratch_types=[
        pltpu.VMEM((KE,), _i32),
        pltpu.VMEM((KE,), _f32),
        pltpu.VMEM((NPT,), _f32),
        pltpu.VMEM_SHARED((NP,), _f32),
    ],
)
def _deg_kernel(col_hbm, ew_hbm, deg_hbm, colv, ewv, zv, acc_sh):
    s = lax.axis_index("s")
    nbase = s * NPT
    _zero_vec(zv, NPT)
    pltpu.sync_copy(zv, acc_sh.at[pl.ds(nbase, NPT)])
    plsc.subcore_barrier()

    e0 = s * EPT

    def chunk(k, _):
        eb = e0 + k * KE
        pltpu.sync_copy(col_hbm.at[pl.ds(eb, KE)], colv)
        pltpu.sync_copy(ew_hbm.at[pl.ds(eb, KE)], ewv)
        pltpu.sync_copy(ewv, acc_sh.at[colv], add=True)
        return 0

    lax.fori_loop(0, EPT // KE, chunk, 0)
    plsc.subcore_barrier()
    pltpu.sync_copy(acc_sh.at[pl.ds(nbase, NPT)], zv)
    pltpu.sync_copy(zv, deg_hbm.at[pl.ds(nbase, NPT)])


def _dis(deg_e):
    def body(d_ref, o_ref):
        o_ref[...] = lax.rsqrt(d_ref[...] + 1.0)

    out = pl.pallas_call(
        body,
        out_shape=jax.ShapeDtypeStruct((NP // 128, 128), _f32),
    )(deg_e.reshape(NP // 128, 128))
    return out.reshape(NP)


def _mm(xp, W1):
    def body(x_ref, w_ref, o_ref):
        o_ref[...] = jnp.dot(
            x_ref[...], w_ref[...],
            preferred_element_type=_f32, precision=lax.Precision.HIGHEST)

    rb = 512
    return pl.pallas_call(
        body,
        grid=(2, NP // rb),
        in_specs=[
            pl.BlockSpec((rb, DF), lambda p, i: (i, 0)),
            pl.BlockSpec((DF, HALF), lambda p, i: (0, p)),
        ],
        out_specs=pl.BlockSpec((rb, HALF), lambda p, i: (p * (NP // rb) + i, 0)),
        out_shape=jax.ShapeDtypeStruct((2 * NP, HALF), _f32),
    )(xp, W1)


@functools.partial(
    pl.kernel,
    out_type=jax.ShapeDtypeStruct((2, NP, HALF), _f32),
    mesh=plsc.VectorSubcoreMesh(
        core_axis_name="c", subcore_axis_name="s", num_cores=2),
    compiler_params=pltpu.CompilerParams(needs_layout_passes=False),
    scratch_types=[
        pltpu.VMEM((KE,), _i32),
        pltpu.VMEM((KE,), _i32),
        pltpu.VMEM((KE,), _i32),
        pltpu.VMEM((KE,), _f32),
        pltpu.VMEM((KE,), _f32),
        pltpu.VMEM((KE, HALF), _f32),
        pltpu.VMEM((CH2, HALF), _f32),
        pltpu.VMEM((CH2, HALF), _f32),
        pltpu.VMEM((HALF,), _f32),
        pltpu.VMEM((NPT,), _f32),
        pltpu.VMEM((NP,), _f32),
        pltpu.VMEM_SHARED((NP, HALF), _f32),
        pltpu.SemaphoreType.DMA,
    ],
)
def _agg1_kernel(row_hbm, col_hbm, ew_hbm, h_hbm, dis_hbm, b1_hbm, out_hbm,
                 rowraw, rowv, colv, ewv, sclv, rows, accv, hv, b1v,
                 disown, dis_full, acc_sh, sem):
    c = lax.axis_index("c")
    s = lax.axis_index("s")
    nbase = s * NPT

    pltpu.sync_copy(dis_hbm.at[pl.ds(nbase, NPT)], disown)
    pltpu.sync_copy(dis_hbm, dis_full)
    pltpu.sync_copy(b1_hbm.at[pl.ds(c * HALF, HALF)], b1v)

    def zrow(i, _):
        for j in range(HALF // L):
            rows[i, pl.ds(j * L, L)] = jnp.zeros((L,), _f32)
        return 0

    lax.fori_loop(0, KE, zrow, 0)
    for t in range(NPT // KE):
        pltpu.sync_copy(rows, acc_sh.at[pl.ds(nbase + t * KE, KE)])
    plsc.subcore_barrier()

    e0 = s * EPT
    shift = c * NP

    def chunk(k, _):
        eb = e0 + k * KE
        pltpu.sync_copy(row_hbm.at[pl.ds(eb, KE)], rowraw)
        pltpu.sync_copy(col_hbm.at[pl.ds(eb, KE)], colv)
        pltpu.sync_copy(ew_hbm.at[pl.ds(eb, KE)], ewv)
        for g in range(KE // L):
            sl = pl.ds(g * L, L)
            r16 = rowraw[sl]
            rowv[sl] = r16 + shift
            sclv[sl] = ewv[sl] * plsc.load_gather(dis_full, [r16])
        pltpu.async_copy(h_hbm.at[rowv], rows, sem).wait()

        def ebody(e, _):
            sb = plsc.load_gather(sclv, [jnp.full((L,), e, _i32)])
            for j in range(HALF // L):
                sl2 = pl.ds(j * L, L)
                rows[e, sl2] = rows[e, sl2] * sb
            return 0

        lax.fori_loop(0, KE, ebody, 0)
        pltpu.sync_copy(rows, acc_sh.at[colv], add=True)
        return 0

    lax.fori_loop(0, EPT // KE, chunk, 0)
    plsc.subcore_barrier()

    def post(t, _):
        node0 = nbase + t * CH2
        pltpu.sync_copy(acc_sh.at[pl.ds(node0, CH2)], accv)
        pltpu.sync_copy(h_hbm.at[pl.ds(shift + node0, CH2)], hv)

        def ebody(e, _):
            db = plsc.load_gather(disown, [jnp.full((L,), t * CH2 + e, _i32)])
            db2 = db * db
            for j in range(HALF // L):
                sl2 = pl.ds(j * L, L)
                v = accv[e, sl2] * db + hv[e, sl2] * db2 + b1v[sl2]
                accv[e, sl2] = _sigmoid16(v)
            return 0

        lax.fori_loop(0, CH2, ebody, 0)
        pltpu.sync_copy(accv, out_hbm.at[c, pl.ds(node0, CH2)])
        return 0

    lax.fori_loop(0, NPT // CH2, post, 0)


@functools.partial(
    pl.kernel,
    out_type=jax.ShapeDtypeStruct((NP,), _f32),
    mesh=plsc.VectorSubcoreMesh(
        core_axis_name="c", subcore_axis_name="s", num_cores=1),
    compiler_params=pltpu.CompilerParams(needs_layout_passes=False),
    scratch_types=[
        pltpu.VMEM((L, HALF), _f32),
        pltpu.VMEM((L, HALF), _f32),
        pltpu.VMEM((HALF,), _f32),
        pltpu.VMEM((HALF,), _f32),
        pltpu.VMEM((8,), _f32),
        pltpu.VMEM((NPT,), _f32),
        pltpu.VMEM((NPT,), _f32),
        pltpu.VMEM((NPT,), _f32),
        pltpu.VMEM((NPT,), _f32),
        pltpu.VMEM((NP,), _f32),
        pltpu.VMEM((KE,), _i32),
        pltpu.VMEM((KE,), _i32),
        pltpu.VMEM((KE,), _f32),
        pltpu.VMEM((KE,), _f32),
        pltpu.VMEM_SHARED((NP,), _f32),
        pltpu.VMEM_SHARED((NP,), _f32),
    ],
)
def _agg2_kernel(h1_hbm, w2_hbm, b2_hbm, row_hbm, col_hbm, ew_hbm, dis_hbm,
                 out_hbm, ha, hb, w2a, w2b, b2v, degv, disown, gv, outv,
                 g_full, rowv, colv, ewv, valv, g_sh, acc_sh):
    s = lax.axis_index("s")
    nbase = s * NPT
    iota = lax.iota(_i32, L)

    pltpu.sync_copy(w2_hbm.at[pl.ds(0, HALF)], w2a)
    pltpu.sync_copy(w2_hbm.at[pl.ds(HALF, HALF)], w2b)
    pltpu.sync_copy(b2_hbm, b2v)
    pltpu.sync_copy(dis_hbm.at[pl.ds(nbase, NPT)], disown)

    _zero_vec(outv, NPT)
    pltpu.sync_copy(outv, acc_sh.at[pl.ds(nbase, NPT)])

    def grp(gi, _):
        off = gi * L
        n0 = nbase + off
        pltpu.sync_copy(h1_hbm.at[0, pl.ds(n0, L)], ha)
        pltpu.sync_copy(h1_hbm.at[1, pl.ds(n0, L)], hb)

        def jb(j, acc):
            jj = jnp.full((L,), j, _i32)
            acc = acc + plsc.load_gather(ha, [iota, jj]) * plsc.load_gather(
                w2a, [jj])
            acc = acc + plsc.load_gather(hb, [iota, jj]) * plsc.load_gather(
                w2b, [jj])
            return acc

        acc = lax.fori_loop(0, HALF, jb, jnp.zeros((L,), _f32))
        gv[pl.ds(off, L)] = acc * plsc.load_gather(disown, [off + iota])
        return 0

    lax.fori_loop(0, NPT // L, grp, 0)
    pltpu.sync_copy(gv, g_sh.at[pl.ds(nbase, NPT)])
    plsc.subcore_barrier()
    pltpu.sync_copy(g_sh, g_full)

    e0 = s * EPT

    def chunk(k, _):
        eb = e0 + k * KE
        pltpu.sync_copy(row_hbm.at[pl.ds(eb, KE)], rowv)
        pltpu.sync_copy(col_hbm.at[pl.ds(eb, KE)], colv)
        pltpu.sync_copy(ew_hbm.at[pl.ds(eb, KE)], ewv)
        for g in range(KE // L):
            sl = pl.ds(g * L, L)
            valv[sl] = ewv[sl] * plsc.load_gather(g_full, [rowv[sl]])
        pltpu.sync_copy(valv, acc_sh.at[colv], add=True)
        return 0

    lax.fori_loop(0, EPT // KE, chunk, 0)
    plsc.subcore_barrier()

    pltpu.sync_copy(acc_sh.at[pl.ds(nbase, NPT)], degv)
    b2b = plsc.load_gather(b2v, [jnp.zeros((L,), _i32)])

    def fin(i, _):
        sl = pl.ds(i * L, L)
        outv[sl] = _sigmoid16(disown[sl] * (degv[sl] + gv[sl]) + b2b)
        return 0

    lax.fori_loop(0, NPT // L, fin, 0)
    pltpu.sync_copy(outv, out_hbm.at[pl.ds(nbase, NPT)])


def kernel(x, edge_index, edge_attr, W1, b1, W2, b2):
    row = edge_index[0].astype(_i32)
    col = edge_index[1].astype(_i32)
    ew = jnp.squeeze(edge_attr, axis=-1)
    xp = jnp.pad(x, ((0, NP - N), (0, 0)))
    b2p = jnp.pad(b2, (0, 8 - b2.shape[0]))
    deg_e = _deg_kernel(col, ew)
    dis = _dis(deg_e)
    h = _mm(xp, W1)
    h1 = _agg1_kernel(row, col, ew, h, dis, b1)
    out = _agg2_kernel(h1, W2[:, 0], b2p, row, col, ew, dis)
    return out[:N, None]


# speedup vs baseline: 9.8835x; 1.9685x over previous
import functools

import jax
import jax.numpy as jnp
from jax import lax
from jax.experimental import pallas as pl
from jax.experimental.pallas import tpu as pltpu
from jax.experimental.pallas import tpu_sc as plsc

N = 10000
NP = 10240
DF = 256
HALF = 128
E = 160000
EP = 163840
NS = 16
L = 16
NPT = NP // NS
EPT = EP // NS
KE = 128
CH2 = 64
CPB = 16
NCH = EPT // KE

_i32 = jnp.int32
_f32 = jnp.float32


def _sigmoid16(v):
    return 1.0 / (1.0 + jnp.exp(-v))


def _zero_vec(ref, n):
    z = jnp.zeros((L,), _f32)

    def body(i, _):
        ref[pl.ds(i * L, L)] = z
        return 0

    lax.fori_loop(0, n // L, body, 0)


@functools.partial(
    pl.kernel,
    out_type=jax.ShapeDtypeStruct((NP,), _f32),
    mesh=plsc.VectorSubcoreMesh(
        core_axis_name="c", subcore_axis_name="s", num_cores=1),
    compiler_params=pltpu.CompilerParams(needs_layout_passes=False),
    scratch_types=[
        pltpu.VMEM((NCH, KE), _i32),
        pltpu.VMEM((NCH, KE), _f32),
        pltpu.VMEM((NPT,), _f32),
        pltpu.VMEM_SHARED((NP,), _f32),
        pltpu.SemaphoreType.DMA,
    ],
)
def _deg_kernel(col2_hbm, ew2_hbm, deg_hbm, colblk, ewblk, zv, acc_sh, sems):
    s = lax.axis_index("s")
    nbase = s * NPT
    _zero_vec(zv, NPT)
    pltpu.sync_copy(zv, acc_sh.at[pl.ds(nbase, NPT)])
    chunk0 = s * NCH
    pltpu.sync_copy(col2_hbm.at[pl.ds(chunk0, NCH)], colblk)
    pltpu.sync_copy(ew2_hbm.at[pl.ds(chunk0, NCH)], ewblk)
    plsc.subcore_barrier()

    qd = 8

    def chunk(k, _):
        pltpu.async_copy(ewblk.at[k], acc_sh.at[colblk.at[k]], sems, add=True)

        @pl.when(k >= qd)
        def _():
            pltpu.make_async_copy(
                ewblk.at[k - qd], acc_sh.at[colblk.at[k - qd]], sems).wait()

        return 0

    lax.fori_loop(0, NCH, chunk, 0)

    def drain(k, _):
        kk = NCH - qd + k
        pltpu.make_async_copy(
            ewblk.at[kk], acc_sh.at[colblk.at[kk]], sems).wait()
        return 0

    lax.fori_loop(0, qd, drain, 0)
    plsc.subcore_barrier()
    pltpu.sync_copy(acc_sh.at[pl.ds(nbase, NPT)], zv)
    pltpu.sync_copy(zv, deg_hbm.at[pl.ds(nbase, NPT)])


def _dis(deg_e):
    def body(d_ref, o_ref):
        o_ref[...] = lax.rsqrt(d_ref[...] + 1.0)

    out = pl.pallas_call(
        body,
        out_shape=jax.ShapeDtypeStruct((NP // 128, 128), _f32),
    )(deg_e.reshape(NP // 128, 128))
    return out.reshape(NP)


def _mm(xp, W1, disc):
    def body(x_ref, w_ref, d_ref, o_ref):
        o_ref[...] = d_ref[...] * jnp.dot(
            x_ref[...], w_ref[...],
            preferred_element_type=_f32, precision=lax.Precision.HIGHEST)

    rb = 512
    return pl.pallas_call(
        body,
        grid=(2, NP // rb),
        in_specs=[
            pl.BlockSpec((rb, DF), lambda p, i: (i, 0)),
            pl.BlockSpec((DF, HALF), lambda p, i: (0, p)),
            pl.BlockSpec((rb, 1), lambda p, i: (i, 0)),
        ],
        out_specs=pl.BlockSpec((rb, HALF), lambda p, i: (p * (NP // rb) + i, 0)),
        out_shape=jax.ShapeDtypeStruct((2 * NP, HALF), _f32),
    )(xp, W1, disc)


@functools.partial(
    pl.kernel,
    out_type=jax.ShapeDtypeStruct((2, NP, HALF), _f32),
    mesh=plsc.VectorSubcoreMesh(
        core_axis_name="c", subcore_axis_name="s", num_cores=2),
    compiler_params=pltpu.CompilerParams(needs_layout_passes=False),
    scratch_types=[
        pltpu.VMEM((CPB, KE), _i32),
        pltpu.VMEM((2, CPB, KE), _i32),
        pltpu.VMEM((CPB, KE), _f32),
        pltpu.VMEM((KE,), _i32),
        pltpu.VMEM((KE,), _i32),
        pltpu.VMEM((KE,), _f32),
        pltpu.VMEM((KE,), _f32),
        pltpu.VMEM((KE, HALF), _f32),
        pltpu.VMEM((KE, HALF), _f32),
        pltpu.VMEM((HALF,), _f32),
        pltpu.VMEM((NPT,), _f32),
        pltpu.VMEM_SHARED((NP, HALF), _f32),
        pltpu.SemaphoreType.DMA,
        pltpu.SemaphoreType.DMA,
        pltpu.SemaphoreType.DMA,
        pltpu.SemaphoreType.DMA,
    ],
)
def _agg1_kernel(row2_hbm, col2_hbm, ew2_hbm, h_hbm, dis_hbm, b1_hbm, out_hbm,
                 rowblk, colblk, ewblk, rvA, rvB, svA, svB, rowsA, rowsB,
                 b1v, disown, acc_sh, semgA, semgB, semsA, semsB):
    c = lax.axis_index("c")
    s = lax.axis_index("s")
    nbase = s * NPT
    shift = c * NP
    chunk0 = s * NCH

    def load_block(blk):
        bp = lax.rem(blk, 2)
        r0 = chunk0 + blk * CPB
        pltpu.sync_copy(row2_hbm.at[pl.ds(r0, CPB)], rowblk)
        pltpu.sync_copy(ew2_hbm.at[pl.ds(r0, CPB)], ewblk)
        pltpu.sync_copy(col2_hbm.at[pl.ds(r0, CPB)], colblk.at[bp])

    def prep(m, rv, sv):
        kr = lax.rem(m, CPB)
        for g in range(KE // L):
            sl = pl.ds(g * L, L)
            rv[sl] = rowblk[kr, sl] + shift
            sv[sl] = ewblk[kr, sl]

    def fire_gather(rv, rows, sem):
        pltpu.async_copy(h_hbm.at[rv], rows, sem)

    def wait_gather(rv, rows, sem):
        pltpu.make_async_copy(h_hbm.at[rv], rows, sem).wait()

    def scale(rows, sv):
        def eb(e, _):
            sb = plsc.load_gather(sv, [jnp.full((L,), e, _i32)])
            for j in range(HALF // L):
                sl = pl.ds(j * L, L)
                rows[e, sl] = rows[e, sl] * sb
            return 0

        lax.fori_loop(0, KE, eb, 0)

    def scat_refs(m, rows):
        bp = lax.rem(m // CPB, 2)
        kr = lax.rem(m, CPB)
        return rows, acc_sh.at[colblk.at[bp, kr]]

    def fire_scatter(m, rows, sem):
        srf, drf = scat_refs(m, rows)
        pltpu.async_copy(srf, drf, sem, add=True)

    def wait_scatter(m, rows, sem):
        srf, drf = scat_refs(m, rows)
        pltpu.make_async_copy(srf, drf, sem).wait()

    pltpu.sync_copy(dis_hbm.at[pl.ds(nbase, NPT)], disown)
    pltpu.sync_copy(b1_hbm.at[pl.ds(c * HALF, HALF)], b1v)

    def zrow(i, _):
        for j in range(HALF // L):
            rowsA[i, pl.ds(j * L, L)] = jnp.zeros((L,), _f32)
        return 0

    lax.fori_loop(0, KE, zrow, 0)
    for t in range(NPT // KE):
        pltpu.sync_copy(rowsA, acc_sh.at[pl.ds(nbase + t * KE, KE)])
    plsc.subcore_barrier()

    load_block(0)
    prep(0, rvA, svA)
    fire_gather(rvA, rowsA, semgA)
    prep(1, rvB, svB)
    fire_gather(rvB, rowsB, semgB)

    def one(m, rv, sv, rows, semg, sems):
        wait_gather(rv, rows, semg)
        scale(rows, sv)
        fire_scatter(m, rows, sems)

        @pl.when(m + 2 <= NCH - 1)
        def _():
            @pl.when(lax.rem(m + 2, CPB) == 0)
            def _():
                load_block((m + 2) // CPB)

            prep(m + 2, rv, sv)
            wait_scatter(m, rows, sems)
            fire_gather(rv, rows, semg)

        @pl.when(m + 2 > NCH - 1)
        def _():
            wait_scatter(m, rows, sems)

    def pair(kk, _):
        m = 2 * kk
        one(m, rvA, svA, rowsA, semgA, semsA)
        one(m + 1, rvB, svB, rowsB, semgB, semsB)
        return 0

    lax.fori_loop(0, NCH // 2, pair, 0)
    plsc.subcore_barrier()

    def post(t, _):
        node0 = nbase + t * CH2
        pltpu.sync_copy(acc_sh.at[pl.ds(node0, CH2)], rowsA.at[pl.ds(0, CH2)])
        pltpu.sync_copy(
            h_hbm.at[pl.ds(shift + node0, CH2)], rowsB.at[pl.ds(0, CH2)])

        def ebody(e, _):
            db = plsc.load_gather(disown, [jnp.full((L,), t * CH2 + e, _i32)])
            for j in range(HALF // L):
                sl2 = pl.ds(j * L, L)
                v = (rowsA[e, sl2] + rowsB[e, sl2]) * db + b1v[sl2]
                rowsA[e, sl2] = _sigmoid16(v)
            return 0

        lax.fori_loop(0, CH2, ebody, 0)
        pltpu.sync_copy(rowsA.at[pl.ds(0, CH2)], out_hbm.at[c, pl.ds(node0, CH2)])
        return 0

    lax.fori_loop(0, NPT // CH2, post, 0)


@functools.partial(
    pl.kernel,
    out_type=jax.ShapeDtypeStruct((NP,), _f32),
    mesh=plsc.VectorSubcoreMesh(
        core_axis_name="c", subcore_axis_name="s", num_cores=1),
    compiler_params=pltpu.CompilerParams(needs_layout_passes=False),
    scratch_types=[
        pltpu.VMEM((L, HALF), _f32),
        pltpu.VMEM((L, HALF), _f32),
        pltpu.VMEM((HALF,), _f32),
        pltpu.VMEM((HALF,), _f32),
        pltpu.VMEM((8,), _f32),
        pltpu.VMEM((NPT,), _f32),
        pltpu.VMEM((NPT,), _f32),
        pltpu.VMEM((NPT,), _f32),
        pltpu.VMEM((NPT,), _f32),
        pltpu.VMEM((NP,), _f32),
        pltpu.VMEM((NCH, KE), _i32),
        pltpu.VMEM((NCH, KE), _i32),
        pltpu.VMEM((NCH, KE), _f32),
        pltpu.VMEM((8, KE), _f32),
        pltpu.VMEM_SHARED((NP,), _f32),
        pltpu.VMEM_SHARED((NP,), _f32),
        pltpu.SemaphoreType.DMA,
    ],
)
def _agg2_kernel(h1_hbm, w2_hbm, b2_hbm, row2_hbm, col2_hbm, ew2_hbm, dis_hbm,
                 out_hbm, ha, hb, w2a, w2b, b2v, degv, disown, gv, outv,
                 g_full, rowblk, colblk, ewblk, valr, g_sh, acc_sh, sems):
    s = lax.axis_index("s")
    nbase = s * NPT
    iota = lax.iota(_i32, L)

    pltpu.sync_copy(w2_hbm.at[pl.ds(0, HALF)], w2a)
    pltpu.sync_copy(w2_hbm.at[pl.ds(HALF, HALF)], w2b)
    pltpu.sync_copy(b2_hbm, b2v)
    pltpu.sync_copy(dis_hbm.at[pl.ds(nbase, NPT)], disown)
    chunk0 = s * NCH
    pltpu.sync_copy(row2_hbm.at[pl.ds(chunk0, NCH)], rowblk)
    pltpu.sync_copy(col2_hbm.at[pl.ds(chunk0, NCH)], colblk)
    pltpu.sync_copy(ew2_hbm.at[pl.ds(chunk0, NCH)], ewblk)

    _zero_vec(outv, NPT)
    pltpu.sync_copy(outv, acc_sh.at[pl.ds(nbase, NPT)])

    def grp(gi, _):
        off = gi * L
        n0 = nbase + off
        pltpu.sync_copy(h1_hbm.at[0, pl.ds(n0, L)], ha)
        pltpu.sync_copy(h1_hbm.at[1, pl.ds(n0, L)], hb)

        def jb(j, acc):
            jj = jnp.full((L,), j, _i32)
            acc = acc + plsc.load_gather(ha, [iota, jj]) * plsc.load_gather(
                w2a, [jj])
            acc = acc + plsc.load_gather(hb, [iota, jj]) * plsc.load_gather(
                w2b, [jj])
            return acc

        acc = lax.fori_loop(0, HALF, jb, jnp.zeros((L,), _f32))
        gv[pl.ds(off, L)] = acc * plsc.load_gather(disown, [off + iota])
        return 0

    lax.fori_loop(0, NPT // L, grp, 0)
    pltpu.sync_copy(gv, g_sh.at[pl.ds(nbase, NPT)])
    plsc.subcore_barrier()
    pltpu.sync_copy(g_sh, g_full)

    qd = 8

    def chunk(k, _):
        kq = lax.rem(k, qd)

        @pl.when(k >= qd)
        def _():
            pltpu.make_async_copy(
                valr.at[kq], acc_sh.at[colblk.at[k - qd]], sems).wait()

        for g in range(KE // L):
            sl = pl.ds(g * L, L)
            valr[kq, sl] = ewblk[k, sl] * plsc.load_gather(
                g_full, [rowblk[k, sl]])
        pltpu.async_copy(valr.at[kq], acc_sh.at[colblk.at[k]], sems, add=True)
        return 0

    lax.fori_loop(0, NCH, chunk, 0)

    def drain(k, _):
        kk = NCH - qd + k
        pltpu.make_async_copy(
            valr.at[lax.rem(kk, qd)], acc_sh.at[colblk.at[kk]], sems).wait()
        return 0

    lax.fori_loop(0, qd, drain, 0)
    plsc.subcore_barrier()

    pltpu.sync_copy(acc_sh.at[pl.ds(nbase, NPT)], degv)
    b2b = plsc.load_gather(b2v, [jnp.zeros((L,), _i32)])

    def fin(i, _):
        sl = pl.ds(i * L, L)
        outv[sl] = _sigmoid16(disown[sl] * (degv[sl] + gv[sl]) + b2b)
        return 0

    lax.fori_loop(0, NPT // L, fin, 0)
    pltpu.sync_copy(outv, out_hbm.at[pl.ds(nbase, NPT)])


def kernel(x, edge_index, edge_attr, W1, b1, W2, b2):
    row = edge_index[0].astype(_i32)
    col = edge_index[1].astype(_i32)
    ew = jnp.squeeze(edge_attr, axis=-1)
    npad = EP - E
    spread = (jnp.arange(npad, dtype=_i32) * 13) % N
    rowp = jnp.concatenate([row, spread]).reshape(-1, KE)
    colp = jnp.concatenate([col, spread]).reshape(-1, KE)
    ewp = jnp.concatenate([ew, jnp.zeros((npad,), _f32)]).reshape(-1, KE)
    xp = jnp.pad(x, ((0, NP - N), (0, 0)))
    b2p = jnp.pad(b2, (0, 8 - b2.shape[0]))
    deg_e = _deg_kernel(colp, ewp)
    dis = _dis(deg_e)
    h = _mm(xp, W1, dis[:, None])
    h1 = _agg1_kernel(rowp, colp, ewp, h, dis, b1)
    out = _agg2_kernel(h1, W2[:, 0], b2p, rowp, colp, ewp, dis)
    return out[:N, None]


# speedup vs baseline: 10.6845x; 1.0810x over previous
import functools

import jax
import jax.numpy as jnp
from jax import lax
from jax.experimental import pallas as pl
from jax.experimental.pallas import tpu as pltpu
from jax.experimental.pallas import tpu_sc as plsc

N = 10000
NP = 10240
DF = 256
HALF = 128
E = 160000
EP = 163840
NS = 16
L = 16
NPT = NP // NS
EPT = EP // NS
KE = 64
CH2 = 64
CPB = 16
NCH = EPT // KE
KD = 128
NCHD = EPT // KD
GD = 64

_i32 = jnp.int32
_f32 = jnp.float32


def _sigmoid16(v):
    return 1.0 / (1.0 + jnp.exp(-v))


def _zero_vec(ref, n):
    z = jnp.zeros((L,), _f32)

    def body(i, _):
        ref[pl.ds(i * L, L)] = z
        return 0

    lax.fori_loop(0, n // L, body, 0)


@functools.partial(
    pl.kernel,
    out_type=jax.ShapeDtypeStruct((NP,), _f32),
    mesh=plsc.VectorSubcoreMesh(
        core_axis_name="c", subcore_axis_name="s", num_cores=1),
    compiler_params=pltpu.CompilerParams(needs_layout_passes=False),
    scratch_types=[
        pltpu.VMEM((NCHD, KD), _i32),
        pltpu.VMEM((NCHD, KD), _f32),
        pltpu.VMEM((NPT,), _f32),
        pltpu.VMEM_SHARED((NP,), _f32),
        pltpu.SemaphoreType.DMA,
    ],
)
def _deg_kernel(col2_hbm, ew2_hbm, deg_hbm, colblk, ewblk, zv, acc_sh, sems):
    s = lax.axis_index("s")
    nbase = s * NPT
    _zero_vec(zv, NPT)
    pltpu.sync_copy(zv, acc_sh.at[pl.ds(nbase, NPT)])
    chunk0 = s * NCHD
    pltpu.sync_copy(col2_hbm.at[pl.ds(chunk0, NCHD)], colblk)
    pltpu.sync_copy(ew2_hbm.at[pl.ds(chunk0, NCHD)], ewblk)
    plsc.subcore_barrier()

    qd = 8

    def chunk(k, _):
        pltpu.async_copy(ewblk.at[k], acc_sh.at[colblk.at[k]], sems, add=True)

        @pl.when(k >= qd)
        def _():
            pltpu.make_async_copy(
                ewblk.at[k - qd], acc_sh.at[colblk.at[k - qd]], sems).wait()

        return 0

    lax.fori_loop(0, NCHD, chunk, 0)

    def drain(k, _):
        kk = NCHD - qd + k
        pltpu.make_async_copy(
            ewblk.at[kk], acc_sh.at[colblk.at[kk]], sems).wait()
        return 0

    lax.fori_loop(0, qd, drain, 0)
    plsc.subcore_barrier()
    pltpu.sync_copy(acc_sh.at[pl.ds(nbase, NPT)], zv)
    pltpu.sync_copy(zv, deg_hbm.at[pl.ds(nbase, NPT)])


def _dis(deg_e):
    def body(d_ref, o_ref):
        o_ref[...] = lax.rsqrt(d_ref[...] + 1.0)

    out = pl.pallas_call(
        body,
        out_shape=jax.ShapeDtypeStruct((NP // 128, 128), _f32),
    )(deg_e.reshape(NP // 128, 128))
    return out.reshape(NP)


def _mm(xp, W1, disc):
    def body(x_ref, w_ref, d_ref, o_ref):
        o_ref[...] = d_ref[...] * jnp.dot(
            x_ref[...], w_ref[...],
            preferred_element_type=_f32, precision=lax.Precision.HIGHEST)

    rb = 512
    return pl.pallas_call(
        body,
        grid=(2, NP // rb),
        in_specs=[
            pl.BlockSpec((rb, DF), lambda p, i: (i, 0)),
            pl.BlockSpec((DF, HALF), lambda p, i: (0, p)),
            pl.BlockSpec((rb, 1), lambda p, i: (i, 0)),
        ],
        out_specs=pl.BlockSpec((rb, HALF), lambda p, i: (p * (NP // rb) + i, 0)),
        out_shape=jax.ShapeDtypeStruct((2 * NP, HALF), _f32),
    )(xp, W1, disc)


@functools.partial(
    pl.kernel,
    out_type=jax.ShapeDtypeStruct((2, NP, HALF), _f32),
    mesh=plsc.VectorSubcoreMesh(
        core_axis_name="c", subcore_axis_name="s", num_cores=2),
    compiler_params=pltpu.CompilerParams(needs_layout_passes=False),
    scratch_types=[
        pltpu.VMEM((CPB, KE), _i32),
        pltpu.VMEM((2, CPB, KE), _i32),
        pltpu.VMEM((CPB, KE), _f32),
        [pltpu.VMEM((KE,), _i32)] * 4,
        [pltpu.VMEM((KE,), _f32)] * 4,
        [pltpu.VMEM((KE, HALF), _f32)] * 4,
        pltpu.VMEM((HALF,), _f32),
        pltpu.VMEM((NPT,), _f32),
        pltpu.VMEM_SHARED((NP, HALF), _f32),
        [pltpu.SemaphoreType.DMA] * 4,
        [pltpu.SemaphoreType.DMA] * 4,
    ],
)
def _agg1_kernel(row2_hbm, col2_hbm, ew2_hbm, h_hbm, dis_hbm, b1_hbm, out_hbm,
                 rowblk, colblk, ewblk, rv, sv, rows,
                 b1v, disown, acc_sh, semg, sems):
    c = lax.axis_index("c")
    s = lax.axis_index("s")
    nbase = s * NPT
    shift = c * NP
    chunk0 = s * NCH

    def load_block(blk):
        bp = lax.rem(blk, 2)
        r0 = chunk0 + blk * CPB
        pltpu.sync_copy(row2_hbm.at[pl.ds(r0, CPB)], rowblk)
        pltpu.sync_copy(ew2_hbm.at[pl.ds(r0, CPB)], ewblk)
        pltpu.sync_copy(col2_hbm.at[pl.ds(r0, CPB)], colblk.at[bp])

    def prep(m, i):
        kr = lax.rem(m, CPB)
        for g in range(KE // L):
            sl = pl.ds(g * L, L)
            rv[i][sl] = rowblk[kr, sl] + shift
            sv[i][sl] = ewblk[kr, sl]

    def fire_gather(m, i):
        pltpu.async_copy(h_hbm.at[rv[i]], rows[i], semg[i])

    def wait_gather(m, i):
        pltpu.make_async_copy(h_hbm.at[rv[i]], rows[i], semg[i]).wait()

    def scale(i):
        def eb(p, _):
            for o in range(2):
                e = 2 * p + o
                sb = plsc.load_gather(sv[i], [jnp.full((L,), e, _i32)])
                for j in range(HALF // L):
                    sl = pl.ds(j * L, L)
                    rows[i][e, sl] = rows[i][e, sl] * sb
            return 0

        lax.fori_loop(0, KE // 2, eb, 0)

    def scat_refs(m, i):
        bp = lax.rem(m // CPB, 2)
        kr = lax.rem(m, CPB)
        return rows[i], acc_sh.at[colblk.at[bp, kr]]

    def fire_scatter(m, i):
        srf, drf = scat_refs(m, i)
        pltpu.async_copy(srf, drf, sems[i], add=True)

    def wait_scatter(m, i):
        srf, drf = scat_refs(m, i)
        pltpu.make_async_copy(srf, drf, sems[i]).wait()

    pltpu.sync_copy(dis_hbm.at[pl.ds(nbase, NPT)], disown)
    pltpu.sync_copy(b1_hbm.at[pl.ds(c * HALF, HALF)], b1v)

    def zrow(i, _):
        for j in range(HALF // L):
            rows[0][i, pl.ds(j * L, L)] = jnp.zeros((L,), _f32)
        return 0

    lax.fori_loop(0, KE, zrow, 0)
    for t in range(NPT // KE):
        pltpu.sync_copy(rows[0], acc_sh.at[pl.ds(nbase + t * KE, KE)])
    plsc.subcore_barrier()

    load_block(0)
    prep(0, 0)
    fire_gather(0, 0)
    prep(1, 1)
    fire_gather(1, 1)

    def sub(m, i):
        wait_gather(m, i)
        scale(i)
        fire_scatter(m, i)
        i2 = (i + 2) % 4

        @pl.when(m + 2 <= NCH - 1)
        def _():
            @pl.when(lax.rem(m + 2, CPB) == 0)
            def _():
                load_block((m + 2) // CPB)

            prep(m + 2, i2)

            @pl.when(m >= 2)
            def _():
                wait_scatter(m - 2, i2)

            fire_gather(m + 2, i2)

        @pl.when(jnp.logical_and(m + 2 > NCH - 1, m >= 2))
        def _():
            wait_scatter(m - 2, i2)

    def quad(q, _):
        m = 4 * q
        for i in range(4):
            sub(m + i, i)
        return 0

    lax.fori_loop(0, NCH // 4, quad, 0)
    wait_scatter(NCH - 2, (NCH - 2) % 4)
    wait_scatter(NCH - 1, (NCH - 1) % 4)
    plsc.subcore_barrier()

    def post(t, _):
        node0 = nbase + t * CH2
        pltpu.sync_copy(acc_sh.at[pl.ds(node0, CH2)], rows[0])
        pltpu.sync_copy(h_hbm.at[pl.ds(shift + node0, CH2)], rows[1])

        def ebody(e, _):
            db = plsc.load_gather(disown, [jnp.full((L,), t * CH2 + e, _i32)])
            for j in range(HALF // L):
                sl2 = pl.ds(j * L, L)
                v = (rows[0][e, sl2] + rows[1][e, sl2]) * db + b1v[sl2]
                rows[0][e, sl2] = _sigmoid16(v)
            return 0

        lax.fori_loop(0, CH2, ebody, 0)
        pltpu.sync_copy(rows[0], out_hbm.at[c, pl.ds(node0, CH2)])
        return 0

    lax.fori_loop(0, NPT // CH2, post, 0)


@functools.partial(
    pl.kernel,
    out_type=jax.ShapeDtypeStruct((NP,), _f32),
    mesh=plsc.VectorSubcoreMesh(
        core_axis_name="c", subcore_axis_name="s", num_cores=1),
    compiler_params=pltpu.CompilerParams(needs_layout_passes=False),
    scratch_types=[
        pltpu.VMEM((GD, HALF), _f32),
        pltpu.VMEM((GD, HALF), _f32),
        pltpu.VMEM((HALF,), _f32),
        pltpu.VMEM((HALF,), _f32),
        pltpu.VMEM((8,), _f32),
        pltpu.VMEM((NPT,), _f32),
        pltpu.VMEM((NPT,), _f32),
        pltpu.VMEM((NPT,), _f32),
        pltpu.VMEM((NPT,), _f32),
        pltpu.VMEM((NP,), _f32),
        pltpu.VMEM((NCHD, KD), _i32),
        pltpu.VMEM((NCHD, KD), _i32),
        pltpu.VMEM((NCHD, KD), _f32),
        pltpu.VMEM((8, KD), _f32),
        pltpu.VMEM_SHARED((NP,), _f32),
        pltpu.VMEM_SHARED((NP,), _f32),
        pltpu.SemaphoreType.DMA,
    ],
)
def _agg2_kernel(h1_hbm, w2_hbm, b2_hbm, row2_hbm, col2_hbm, ew2_hbm, dis_hbm,
                 out_hbm, ha, hb, w2a, w2b, b2v, degv, disown, gv, outv,
                 g_full, rowblk, colblk, ewblk, valr, g_sh, acc_sh, sems):
    s = lax.axis_index("s")
    nbase = s * NPT
    iota = lax.iota(_i32, L)

    pltpu.sync_copy(w2_hbm.at[pl.ds(0, HALF)], w2a)
    pltpu.sync_copy(w2_hbm.at[pl.ds(HALF, HALF)], w2b)
    pltpu.sync_copy(b2_hbm, b2v)
    pltpu.sync_copy(dis_hbm.at[pl.ds(nbase, NPT)], disown)
    chunk0 = s * NCHD
    pltpu.sync_copy(row2_hbm.at[pl.ds(chunk0, NCHD)], rowblk)
    pltpu.sync_copy(col2_hbm.at[pl.ds(chunk0, NCHD)], colblk)
    pltpu.sync_copy(ew2_hbm.at[pl.ds(chunk0, NCHD)], ewblk)

    _zero_vec(outv, NPT)
    pltpu.sync_copy(outv, acc_sh.at[pl.ds(nbase, NPT)])

    def blk(t, _):
        off0 = t * GD
        n0 = nbase + off0
        pltpu.sync_copy(h1_hbm.at[0, pl.ds(n0, GD)], ha)
        pltpu.sync_copy(h1_hbm.at[1, pl.ds(n0, GD)], hb)

        def grp(gi, _):
            off = gi * L

            def jb(jq, acc):
                for o in range(4):
                    j = 4 * jq + o
                    jj = jnp.full((L,), j, _i32)
                    acc = acc + plsc.load_gather(
                        ha, [off + iota, jj]) * plsc.load_gather(w2a, [jj])
                    acc = acc + plsc.load_gather(
                        hb, [off + iota, jj]) * plsc.load_gather(w2b, [jj])
                return acc

            acc = lax.fori_loop(0, HALF // 4, jb, jnp.zeros((L,), _f32))
            o2 = off0 + off
            gv[pl.ds(o2, L)] = acc * plsc.load_gather(disown, [o2 + iota])
            return 0

        lax.fori_loop(0, GD // L, grp, 0)
        return 0

    lax.fori_loop(0, NPT // GD, blk, 0)
    pltpu.sync_copy(gv, g_sh.at[pl.ds(nbase, NPT)])
    plsc.subcore_barrier()
    pltpu.sync_copy(g_sh, g_full)

    qd = 8

    def chunk(k, _):
        kq = lax.rem(k, qd)

        @pl.when(k >= qd)
        def _():
            pltpu.make_async_copy(
                valr.at[kq], acc_sh.at[colblk.at[k - qd]], sems).wait()

        for g in range(KD // L):
            sl = pl.ds(g * L, L)
            valr[kq, sl] = ewblk[k, sl] * plsc.load_gather(
                g_full, [rowblk[k, sl]])
        pltpu.async_copy(valr.at[kq], acc_sh.at[colblk.at[k]], sems, add=True)
        return 0

    lax.fori_loop(0, NCHD, chunk, 0)

    def drain(k, _):
        kk = NCHD - qd + k
        pltpu.make_async_copy(
            valr.at[lax.rem(kk, qd)], acc_sh.at[colblk.at[kk]], sems).wait()
        return 0

    lax.fori_loop(0, qd, drain, 0)
    plsc.subcore_barrier()

    pltpu.sync_copy(acc_sh.at[pl.ds(nbase, NPT)], degv)
    b2b = plsc.load_gather(b2v, [jnp.zeros((L,), _i32)])

    def fin(i, _):
        sl = pl.ds(i * L, L)
        outv[sl] = _sigmoid16(disown[sl] * (degv[sl] + gv[sl]) + b2b)
        return 0

    lax.fori_loop(0, NPT // L, fin, 0)
    pltpu.sync_copy(outv, out_hbm.at[pl.ds(nbase, NPT)])


def kernel(x, edge_index, edge_attr, W1, b1, W2, b2):
    row = edge_index[0].astype(_i32)
    col = edge_index[1].astype(_i32)
    ew = jnp.squeeze(edge_attr, axis=-1)
    npad = EP - E
    spread = (jnp.arange(npad, dtype=_i32) * 13) % N
    rowp = jnp.concatenate([row, spread])
    colp = jnp.concatenate([col, spread])
    ewp = jnp.concatenate([ew, jnp.zeros((npad,), _f32)])
    r64, c64, w64 = (a.reshape(-1, KE) for a in (rowp, colp, ewp))
    r128, c128, w128 = (a.reshape(-1, KD) for a in (rowp, colp, ewp))
    xp = jnp.pad(x, ((0, NP - N), (0, 0)))
    b2p = jnp.pad(b2, (0, 8 - b2.shape[0]))
    deg_e = _deg_kernel(c128, w128)
    dis = _dis(deg_e)
    h = _mm(xp, W1, dis[:, None])
    h1 = _agg1_kernel(r64, c64, w64, h, dis, b1)
    out = _agg2_kernel(h1, W2[:, 0], b2p, r128, c128, w128, dis)
    return out[:N, None]


# speedup vs baseline: 10.8548x; 1.0159x over previous
import functools

import jax
import jax.numpy as jnp
from jax import lax
from jax.experimental import pallas as pl
from jax.experimental.pallas import tpu as pltpu
from jax.experimental.pallas import tpu_sc as plsc

N = 10000
NP = 10240
DF = 256
HALF = 128
E = 160000
EP = 163840
NS = 16
L = 16
NPT = NP // NS
EPT = EP // NS
KE = 64
CH2 = 64
CPB = 16
NCH = EPT // KE
KD = 128
NCHD = EPT // KD
GD = 64

_i32 = jnp.int32
_f32 = jnp.float32


def _sigmoid16(v):
    return 1.0 / (1.0 + jnp.exp(-v))


def _zero_vec(ref, n):
    z = jnp.zeros((L,), _f32)

    def body(i, _):
        ref[pl.ds(i * L, L)] = z
        return 0

    lax.fori_loop(0, n // L, body, 0)


@functools.partial(
    pl.kernel,
    out_type=jax.ShapeDtypeStruct((NP,), _f32),
    mesh=plsc.VectorSubcoreMesh(
        core_axis_name="c", subcore_axis_name="s", num_cores=1),
    compiler_params=pltpu.CompilerParams(needs_layout_passes=False),
    scratch_types=[
        pltpu.VMEM((NCHD, KD), _i32),
        pltpu.VMEM((NCHD, KD), _f32),
        pltpu.VMEM((NPT,), _f32),
        pltpu.VMEM_SHARED((NP,), _f32),
        pltpu.SemaphoreType.DMA,
    ],
)
def _deg_kernel(col2_hbm, ew2_hbm, deg_hbm, colblk, ewblk, zv, acc_sh, sems):
    s = lax.axis_index("s")
    nbase = s * NPT
    _zero_vec(zv, NPT)
    pltpu.sync_copy(zv, acc_sh.at[pl.ds(nbase, NPT)])
    chunk0 = s * NCHD
    pltpu.sync_copy(col2_hbm.at[pl.ds(chunk0, NCHD)], colblk)
    pltpu.sync_copy(ew2_hbm.at[pl.ds(chunk0, NCHD)], ewblk)
    plsc.subcore_barrier()

    qd = 8

    def chunk(k, _):
        pltpu.async_copy(ewblk.at[k], acc_sh.at[colblk.at[k]], sems, add=True)

        @pl.when(k >= qd)
        def _():
            pltpu.make_async_copy(
                ewblk.at[k - qd], acc_sh.at[colblk.at[k - qd]], sems).wait()

        return 0

    lax.fori_loop(0, NCHD, chunk, 0)

    def drain(k, _):
        kk = NCHD - qd + k
        pltpu.make_async_copy(
            ewblk.at[kk], acc_sh.at[colblk.at[kk]], sems).wait()
        return 0

    lax.fori_loop(0, qd, drain, 0)
    plsc.subcore_barrier()
    pltpu.sync_copy(acc_sh.at[pl.ds(nbase, NPT)], zv)
    pltpu.sync_copy(zv, deg_hbm.at[pl.ds(nbase, NPT)])


def _dis(deg_e):
    def body(d_ref, o_ref):
        o_ref[...] = lax.rsqrt(d_ref[...] + 1.0)

    out = pl.pallas_call(
        body,
        out_shape=jax.ShapeDtypeStruct((NP // 128, 128), _f32),
    )(deg_e.reshape(NP // 128, 128))
    return out.reshape(NP)


def _mm(xp, W1, disc):
    def body(x_ref, w_ref, d_ref, o_ref):
        o_ref[...] = d_ref[...] * jnp.dot(
            x_ref[...], w_ref[...], preferred_element_type=_f32)

    rb = 512
    return pl.pallas_call(
        body,
        grid=(2, NP // rb),
        in_specs=[
            pl.BlockSpec((rb, DF), lambda p, i: (i, 0)),
            pl.BlockSpec((DF, HALF), lambda p, i: (0, p)),
            pl.BlockSpec((rb, 1), lambda p, i: (i, 0)),
        ],
        out_specs=pl.BlockSpec((rb, HALF), lambda p, i: (p * (NP // rb) + i, 0)),
        out_shape=jax.ShapeDtypeStruct((2 * NP, HALF), _f32),
    )(xp, W1, disc)


@functools.partial(
    pl.kernel,
    out_type=jax.ShapeDtypeStruct((2, NP, HALF), _f32),
    mesh=plsc.VectorSubcoreMesh(
        core_axis_name="c", subcore_axis_name="s", num_cores=2),
    compiler_params=pltpu.CompilerParams(needs_layout_passes=False),
    scratch_types=[
        pltpu.VMEM((CPB, KE), _i32),
        pltpu.VMEM((2, CPB, KE), _i32),
        pltpu.VMEM((CPB, KE), _f32),
        [pltpu.VMEM((KE,), _i32)] * 4,
        [pltpu.VMEM((KE,), _f32)] * 4,
        [pltpu.VMEM((KE, HALF), _f32)] * 4,
        pltpu.VMEM((HALF,), _f32),
        pltpu.VMEM((NPT,), _f32),
        pltpu.VMEM_SHARED((NP, HALF), _f32),
        [pltpu.SemaphoreType.DMA] * 4,
        [pltpu.SemaphoreType.DMA] * 4,
    ],
)
def _agg1_kernel(row2_hbm, col2_hbm, ew2_hbm, h_hbm, dis_hbm, b1_hbm, out_hbm,
                 rowblk, colblk, ewblk, rv, sv, rows,
                 b1v, disown, acc_sh, semg, sems):
    c = lax.axis_index("c")
    s = lax.axis_index("s")
    nbase = s * NPT
    shift = c * NP
    chunk0 = s * NCH

    def load_block(blk):
        bp = lax.rem(blk, 2)
        r0 = chunk0 + blk * CPB
        pltpu.sync_copy(row2_hbm.at[pl.ds(r0, CPB)], rowblk)
        pltpu.sync_copy(ew2_hbm.at[pl.ds(r0, CPB)], ewblk)
        pltpu.sync_copy(col2_hbm.at[pl.ds(r0, CPB)], colblk.at[bp])

    def prep(m, i):
        kr = lax.rem(m, CPB)
        for g in range(KE // L):
            sl = pl.ds(g * L, L)
            rv[i][sl] = rowblk[kr, sl] + shift
            sv[i][sl] = ewblk[kr, sl]

    def fire_gather(m, i):
        pltpu.async_copy(h_hbm.at[rv[i]], rows[i], semg[i])

    def wait_gather(m, i):
        pltpu.make_async_copy(h_hbm.at[rv[i]], rows[i], semg[i]).wait()

    def scale(i):
        def eb(p, _):
            for o in range(2):
                e = 2 * p + o
                sb = plsc.load_gather(sv[i], [jnp.full((L,), e, _i32)])
                for j in range(HALF // L):
                    sl = pl.ds(j * L, L)
                    rows[i][e, sl] = rows[i][e, sl] * sb
            return 0

        lax.fori_loop(0, KE // 2, eb, 0)

    def scat_refs(m, i):
        bp = lax.rem(m // CPB, 2)
        kr = lax.rem(m, CPB)
        return rows[i], acc_sh.at[colblk.at[bp, kr]]

    def fire_scatter(m, i):
        srf, drf = scat_refs(m, i)
        pltpu.async_copy(srf, drf, sems[i], add=True)

    def wait_scatter(m, i):
        srf, drf = scat_refs(m, i)
        pltpu.make_async_copy(srf, drf, sems[i]).wait()

    pltpu.sync_copy(dis_hbm.at[pl.ds(nbase, NPT)], disown)
    pltpu.sync_copy(b1_hbm.at[pl.ds(c * HALF, HALF)], b1v)

    def zrow(i, _):
        for j in range(HALF // L):
            rows[0][i, pl.ds(j * L, L)] = jnp.zeros((L,), _f32)
        return 0

    lax.fori_loop(0, KE, zrow, 0)
    for t in range(NPT // KE):
        pltpu.sync_copy(rows[0], acc_sh.at[pl.ds(nbase + t * KE, KE)])
    plsc.subcore_barrier()

    load_block(0)
    prep(0, 0)
    fire_gather(0, 0)
    prep(1, 1)
    fire_gather(1, 1)

    def sub(m, i):
        wait_gather(m, i)
        scale(i)
        fire_scatter(m, i)
        i2 = (i + 2) % 4

        @pl.when(m + 2 <= NCH - 1)
        def _():
            @pl.when(lax.rem(m + 2, CPB) == 0)
            def _():
                load_block((m + 2) // CPB)

            prep(m + 2, i2)

            @pl.when(m >= 2)
            def _():
                wait_scatter(m - 2, i2)

            fire_gather(m + 2, i2)

        @pl.when(jnp.logical_and(m + 2 > NCH - 1, m >= 2))
        def _():
            wait_scatter(m - 2, i2)

    def quad(q, _):
        m = 4 * q
        for i in range(4):
            sub(m + i, i)
        return 0

    lax.fori_loop(0, NCH // 4, quad, 0)
    wait_scatter(NCH - 2, (NCH - 2) % 4)
    wait_scatter(NCH - 1, (NCH - 1) % 4)
    plsc.subcore_barrier()

    def post(t, _):
        node0 = nbase + t * CH2
        pltpu.sync_copy(acc_sh.at[pl.ds(node0, CH2)], rows[0])
        pltpu.sync_copy(h_hbm.at[pl.ds(shift + node0, CH2)], rows[1])

        def ebody(e, _):
            db = plsc.load_gather(disown, [jnp.full((L,), t * CH2 + e, _i32)])
            for j in range(HALF // L):
                sl2 = pl.ds(j * L, L)
                v = (rows[0][e, sl2] + rows[1][e, sl2]) * db + b1v[sl2]
                rows[0][e, sl2] = _sigmoid16(v)
            return 0

        lax.fori_loop(0, CH2, ebody, 0)
        pltpu.sync_copy(rows[0], out_hbm.at[c, pl.ds(node0, CH2)])
        return 0

    lax.fori_loop(0, NPT // CH2, post, 0)


@functools.partial(
    pl.kernel,
    out_type=[jax.ShapeDtypeStruct((2, NP), _f32),
              jax.ShapeDtypeStruct((NP,), _f32)],
    mesh=plsc.VectorSubcoreMesh(
        core_axis_name="c", subcore_axis_name="s", num_cores=2),
    compiler_params=pltpu.CompilerParams(needs_layout_passes=False),
    scratch_types=[
        pltpu.VMEM((GD, HALF), _f32),
        pltpu.VMEM((GD, HALF), _f32),
        pltpu.VMEM((HALF,), _f32),
        pltpu.VMEM((HALF,), _f32),
        pltpu.VMEM((NPT,), _f32),
        pltpu.VMEM((NPT,), _f32),
        pltpu.VMEM((NPT,), _f32),
        pltpu.VMEM((NP,), _f32),
        pltpu.VMEM((NCHD, KD), _i32),
        pltpu.VMEM((NCHD, KD), _i32),
        pltpu.VMEM((NCHD, KD), _f32),
        pltpu.VMEM((8, KD), _f32),
        pltpu.VMEM_SHARED((NP,), _f32),
        pltpu.VMEM_SHARED((NP,), _f32),
        pltpu.SemaphoreType.DMA,
    ],
)
def _agg2_kernel(h1_hbm, w2_hbm, row2_hbm, col2_hbm, ew2_hbm, dis_hbm,
                 z2p_hbm, g2_hbm, ha, hb, w2a, w2b, degv, disown, gv,
                 g_full, rowblk, colblk, ewblk, valr, g_sh, acc_sh, sems):
    cc = lax.axis_index("c")
    s = lax.axis_index("s")
    nbase = s * NPT
    iota = lax.iota(_i32, L)

    pltpu.sync_copy(w2_hbm.at[pl.ds(0, HALF)], w2a)
    pltpu.sync_copy(w2_hbm.at[pl.ds(HALF, HALF)], w2b)
    pltpu.sync_copy(dis_hbm.at[pl.ds(nbase, NPT)], disown)
    chunk0 = s * NCHD
    pltpu.sync_copy(row2_hbm.at[pl.ds(chunk0, NCHD)], rowblk)
    pltpu.sync_copy(col2_hbm.at[pl.ds(chunk0, NCHD)], colblk)
    pltpu.sync_copy(ew2_hbm.at[pl.ds(chunk0, NCHD)], ewblk)

    _zero_vec(degv, NPT)
    pltpu.sync_copy(degv, acc_sh.at[pl.ds(nbase, NPT)])

    def blk(t, _):
        off0 = t * GD
        n0 = nbase + off0
        pltpu.sync_copy(h1_hbm.at[0, pl.ds(n0, GD)], ha)
        pltpu.sync_copy(h1_hbm.at[1, pl.ds(n0, GD)], hb)

        def grp(gi, _):
            off = gi * L

            def jb(jq, acc):
                for o in range(4):
                    j = 4 * jq + o
                    jj = jnp.full((L,), j, _i32)
                    acc = acc + plsc.load_gather(
                        ha, [off + iota, jj]) * plsc.load_gather(w2a, [jj])
                    acc = acc + plsc.load_gather(
                        hb, [off + iota, jj]) * plsc.load_gather(w2b, [jj])
                return acc

            acc = lax.fori_loop(0, HALF // 4, jb, jnp.zeros((L,), _f32))
            o2 = off0 + off
            gv[pl.ds(o2, L)] = acc * plsc.load_gather(disown, [o2 + iota])
            return 0

        lax.fori_loop(0, GD // L, grp, 0)
        return 0

    lax.fori_loop(0, NPT // GD, blk, 0)
    pltpu.sync_copy(gv, g_sh.at[pl.ds(nbase, NPT)])
    plsc.subcore_barrier()
    pltpu.sync_copy(g_sh, g_full)

    qd = 8
    khalf = NCHD // 2
    k0 = cc * khalf

    def chunk(k2, _):
        k = k0 + k2
        kq = lax.rem(k2, qd)

        @pl.when(k2 >= qd)
        def _():
            pltpu.make_async_copy(
                valr.at[kq], acc_sh.at[colblk.at[k - qd]], sems).wait()

        for g in range(KD // L):
            sl = pl.ds(g * L, L)
            valr[kq, sl] = ewblk[k, sl] * plsc.load_gather(
                g_full, [rowblk[k, sl]])
        pltpu.async_copy(valr.at[kq], acc_sh.at[colblk.at[k]], sems, add=True)
        return 0

    lax.fori_loop(0, khalf, chunk, 0)

    def drain(k2, _):
        kk = k0 + khalf - qd + k2
        pltpu.make_async_copy(
            valr.at[lax.rem(khalf - qd + k2, qd)],
            acc_sh.at[colblk.at[kk]], sems).wait()
        return 0

    lax.fori_loop(0, qd, drain, 0)
    plsc.subcore_barrier()

    pltpu.sync_copy(acc_sh.at[pl.ds(nbase, NPT)], degv)
    pltpu.sync_copy(degv, z2p_hbm.at[cc, pl.ds(nbase, NPT)])

    @pl.when(cc == 0)
    def _():
        pltpu.sync_copy(gv, g2_hbm.at[pl.ds(nbase, NPT)])


def _fin(z2p, g2, dis, b2):
    def body(p_ref, g_ref, d_ref, b_ref, o_ref):
        z = d_ref[...] * (p_ref[0] + p_ref[1] + g_ref[...]) + b_ref[...]
        o_ref[...] = jax.nn.sigmoid(z)

    out = pl.pallas_call(
        body,
        out_shape=jax.ShapeDtypeStruct((NP // 128, 128), _f32),
    )(z2p.reshape(2, NP // 128, 128), g2.reshape(NP // 128, 128),
      dis.reshape(NP // 128, 128), b2.reshape(1, 1))
    return out.reshape(NP)


def kernel(x, edge_index, edge_attr, W1, b1, W2, b2):
    row = edge_index[0].astype(_i32)
    col = edge_index[1].astype(_i32)
    ew = jnp.squeeze(edge_attr, axis=-1)
    npad = EP - E
    spread = (jnp.arange(npad, dtype=_i32) * 13) % N
    rowp = jnp.concatenate([row, spread])
    colp = jnp.concatenate([col, spread])
    ewp = jnp.concatenate([ew, jnp.zeros((npad,), _f32)])
    r64, c64, w64 = (a.reshape(-1, KE) for a in (rowp, colp, ewp))
    r128, c128, w128 = (a.reshape(-1, KD) for a in (rowp, colp, ewp))
    xp = jnp.pad(x, ((0, NP - N), (0, 0)))
    deg_e = _deg_kernel(c128, w128)
    dis = _dis(deg_e)
    h = _mm(xp, W1, dis[:, None])
    h1 = _agg1_kernel(r64, c64, w64, h, dis, b1)
    z2p, g2 = _agg2_kernel(h1, W2[:, 0], r128, c128, w128, dis)
    out = _fin(z2p, g2, dis, b2)
    return out[:N, None]
